# Initial kernel scaffold; baseline (speedup 1.0000x reference)
import jax
import jax.numpy as jnp
from jax.experimental import pallas as pl


def kernel(x, table):
    raise NotImplementedError("write your pallas kernel here")


# speedup vs baseline: 7.8808x; 7.8808x over previous
import functools
import math

import jax
import jax.numpy as jnp
from jax import lax
from jax.experimental import pallas as pl
from jax.experimental.pallas import tpu as pltpu
from jax.experimental.pallas import tpu_sc as plsc

D = 128
SCALE = math.sqrt(float(D))
NC = 2
NS = 16
NW = NC * NS
B = 1024 * 200
B_PER_W = B // NW
CHUNK = 128
NCHUNK = B_PER_W // CHUNK
NBUF = 2


def _emb_body(x_hbm, table_hbm, out_hbm,
              idx_v, in_buf0, in_buf1, out_buf0, out_buf1,
              in_sem0, in_sem1, out_sem0, out_sem1):
    in_bufs = (in_buf0, in_buf1)
    out_bufs = (out_buf0, out_buf1)
    in_sems = (in_sem0, in_sem1)
    out_sems = (out_sem0, out_sem1)

    c = lax.axis_index("c")
    s = lax.axis_index("s")
    wid = s * NC + c
    base = wid * B_PER_W

    pltpu.sync_copy(x_hbm.at[wid], idx_v)

    def gather(chunk, b):
        return pltpu.make_async_copy(
            table_hbm.at[idx_v.at[chunk]], in_bufs[b], in_sems[b])

    def put(chunk, b):
        return pltpu.make_async_copy(
            out_bufs[b], out_hbm.at[pl.ds(base + chunk * CHUNK, CHUNK)],
            out_sems[b])

    for b in range(NBUF):
        gather(b, b).start()

    def slot(chunk, b):
        gather(chunk, b).wait()
        @pl.when(chunk >= NBUF)
        def _():
            put(chunk - NBUF, b).wait()

        def row(r, carry):
            for cc in range(D // 16):
                out_bufs[b][r, pl.ds(cc * 16, 16)] = (
                    in_bufs[b][r, pl.ds(cc * 16, 16)] * SCALE)
            return carry
        lax.fori_loop(0, CHUNK, row, 0)

        @pl.when(chunk + NBUF < NCHUNK)
        def _():
            gather(chunk + NBUF, b).start()

        put(chunk, b).start()

    def outer(j, carry):
        for b in range(NBUF):
            slot(j * NBUF + b, b)
        return carry
    lax.fori_loop(0, NCHUNK // NBUF, outer, 0)

    for b in range(NBUF):
        put(NCHUNK - NBUF + b, b).wait()


@functools.partial(jax.jit, static_argnames=())
def kernel(x, table):
    mesh = plsc.VectorSubcoreMesh(core_axis_name="c", subcore_axis_name="s")
    run = functools.partial(
        pl.kernel,
        mesh=mesh,
        out_type=jax.ShapeDtypeStruct((B, D), jnp.float32),
        scratch_types=[
            pltpu.VMEM((NCHUNK, CHUNK), jnp.int32),
            pltpu.VMEM((CHUNK, D), jnp.float32),
            pltpu.VMEM((CHUNK, D), jnp.float32),
            pltpu.VMEM((CHUNK, D), jnp.float32),
            pltpu.VMEM((CHUNK, D), jnp.float32),
            pltpu.SemaphoreType.DMA,
            pltpu.SemaphoreType.DMA,
            pltpu.SemaphoreType.DMA,
            pltpu.SemaphoreType.DMA,
        ],
    )(_emb_body)
    x3d = x.reshape(NW, NCHUNK, CHUNK).astype(jnp.int32)
    out = run(x3d, table)
    return out.reshape(x.shape[0], x.shape[1], D)


# speedup vs baseline: 7.9292x; 1.0061x over previous
import functools
import math

import jax
import jax.numpy as jnp
from jax import lax
from jax.experimental import pallas as pl
from jax.experimental.pallas import tpu as pltpu
from jax.experimental.pallas import tpu_sc as plsc

D = 128
SCALE = math.sqrt(float(D))
NC = 2
NS = 16
NW = NC * NS
B = 1024 * 200
B_PER_W = B // NW
CHUNK = 128
NCHUNK = B_PER_W // CHUNK
NBUF = 3


def _emb_body(x_hbm, table_hbm, out_hbm, idx_v, *scratch):
    in_bufs = scratch[:NBUF]
    out_bufs = scratch[NBUF:2 * NBUF]
    in_sems = scratch[2 * NBUF:3 * NBUF]
    out_sems = scratch[3 * NBUF:4 * NBUF]

    c = lax.axis_index("c")
    s = lax.axis_index("s")
    wid = s * NC + c
    base = wid * B_PER_W

    pltpu.sync_copy(x_hbm.at[wid], idx_v)

    def gather(chunk, b):
        return pltpu.make_async_copy(
            table_hbm.at[idx_v.at[chunk]], in_bufs[b], in_sems[b])

    def put(chunk, b):
        return pltpu.make_async_copy(
            out_bufs[b], out_hbm.at[pl.ds(base + chunk * CHUNK, CHUNK)],
            out_sems[b])

    for b in range(NBUF):
        gather(jnp.int32(b), b).start()

    def slot(chunk, b):
        gather(chunk, b).wait()
        @pl.when(chunk >= NBUF)
        def _():
            put(chunk - NBUF, b).wait()

        def row(r, carry):
            for cc in range(D // 16):
                out_bufs[b][r, pl.ds(cc * 16, 16)] = (
                    in_bufs[b][r, pl.ds(cc * 16, 16)] * SCALE)
            return carry
        lax.fori_loop(0, CHUNK, row, 0)

        @pl.when(chunk + NBUF < NCHUNK)
        def _():
            gather(chunk + NBUF, b).start()

        put(chunk, b).start()

    full, rem = divmod(NCHUNK, NBUF)

    def outer(j, carry):
        for b in range(NBUF):
            slot(j * NBUF + b, b)
        return carry
    lax.fori_loop(0, full, outer, 0)
    for r in range(rem):
        slot(jnp.int32(full * NBUF + r), r)

    for chunk in range(NCHUNK - NBUF, NCHUNK):
        put(jnp.int32(chunk), chunk % NBUF).wait()


def kernel(x, table):
    mesh = plsc.VectorSubcoreMesh(core_axis_name="c", subcore_axis_name="s")
    scratch = (
        [pltpu.VMEM((NCHUNK, CHUNK), jnp.int32)]
        + [pltpu.VMEM((CHUNK, D), jnp.float32) for _ in range(2 * NBUF)]
        + [pltpu.SemaphoreType.DMA for _ in range(2 * NBUF)]
    )
    run = functools.partial(
        pl.kernel,
        mesh=mesh,
        out_type=jax.ShapeDtypeStruct((B, D), jnp.float32),
        scratch_types=scratch,
    )(_emb_body)
    x3d = x.reshape(NW, NCHUNK, CHUNK).astype(jnp.int32)
    out = run(x3d, table)
    return out.reshape(x.shape[0], x.shape[1], D)
